# baseline (speedup 1.0000x reference)
import functools

import jax
import jax.numpy as jnp
from jax import lax
from jax.experimental import pallas as pl
from jax.experimental.pallas import tpu as pltpu
from jax.experimental.pallas import tpu_sc as plsc

F32 = jnp.float32
I32 = jnp.int32
ROW_TILE = 256

NCORES = 2
NSUB = 16
HALF = 64
D = 128
BROWS = 4480
TROWS = 4608
ZR = 32
CHUNK = 64


def _ceil_to(a, m):
    return (a + m - 1) // m * m


def _pad2(x, rows, cols):
    pr, pc = rows - x.shape[0], cols - x.shape[1]
    if pr == 0 and pc == 0:
        return x
    return jnp.pad(x, ((0, pr), (0, pc)))


def _silu(v):
    return v / (1.0 + jnp.exp(-v))


def _finalize_stats(s, q, n, eps=1e-5):
    mu = (s / n).reshape(-1)
    var = (q / n).reshape(-1) - mu * mu
    rstd = 1.0 / jnp.sqrt(jnp.maximum(var, 0.0) + eps)
    return mu, rstd


def _mm(x, w, b, *, stats=False, norm=None, split=False):
    R, din = x.shape
    dout = w.shape[1]
    rt = min(ROW_TILE, _ceil_to(R, 8))
    rp = _ceil_to(R, rt)
    dinp = _ceil_to(din, 8)
    xp = _pad2(x, rp, dinp)
    wp = _pad2(w, dinp, dout)
    b2 = b.reshape(1, dout)
    grid = (rp // rt,)

    args = [xp]
    in_specs = [pl.BlockSpec((rt, dinp), lambda i: (i, 0))]
    if norm is not None:
        for v in norm:
            args.append(_pad2(v.reshape(1, din), 1, dinp))
            in_specs.append(pl.BlockSpec((1, dinp), lambda i: (0, 0)))
    args += [wp, b2]
    in_specs += [pl.BlockSpec((dinp, dout), lambda i: (0, 0)),
                 pl.BlockSpec((1, dout), lambda i: (0, 0))]

    if split:
        out_shape = [jax.ShapeDtypeStruct((2, rp, dout // 2), F32)]
        out_specs = [pl.BlockSpec((2, rt, dout // 2), lambda i: (0, i, 0))]
    else:
        out_shape = [jax.ShapeDtypeStruct((rp, dout), F32)]
        out_specs = [pl.BlockSpec((rt, dout), lambda i: (i, 0))]
    if stats:
        out_shape += [jax.ShapeDtypeStruct((1, dout), F32)] * 2
        out_specs += [pl.BlockSpec((1, dout), lambda i: (0, 0))] * 2

    def body(*refs):
        if norm is not None:
            x_ref, mu_ref, rstd_ref, g_ref, be_ref, w_ref, b_ref = refs[:7]
            rest = refs[7:]
        else:
            x_ref, w_ref, b_ref = refs[:3]
            rest = refs[3:]
        xv = x_ref[...]
        if norm is not None:
            xv = (xv - mu_ref[...]) * rstd_ref[...] * g_ref[...] + be_ref[...]
            xv = _silu(xv)
        t = jnp.dot(xv, w_ref[...], preferred_element_type=F32) + b_ref[...]
        if stats:
            out_ref, s_ref, q_ref = rest
        else:
            (out_ref,) = rest
        if split:
            h = dout // 2
            out_ref[0] = t[:, :h]
            out_ref[1] = t[:, h:]
        else:
            out_ref[...] = t
        if stats:
            i = pl.program_id(0)
            rows = i * rt + lax.broadcasted_iota(I32, (rt, dout), 0)
            tm = jnp.where(rows < R, t, 0.0)

            @pl.when(i == 0)
            def _():
                s_ref[...] = jnp.zeros_like(s_ref)
                q_ref[...] = jnp.zeros_like(q_ref)

            s_ref[...] += jnp.sum(tm, axis=0, keepdims=True)
            q_ref[...] += jnp.sum(tm * tm, axis=0, keepdims=True)

    outs = pl.pallas_call(body, grid=grid, in_specs=in_specs,
                          out_specs=out_specs, out_shape=out_shape)(*args)
    if stats:
        return (outs[0] if split else outs[0][:R]), outs[1], outs[2]
    return outs[0] if split else outs[0][:R]


def _mm4_split(x, ws, bs):
    R, d = x.shape
    h = d // 2
    rt = min(ROW_TILE, _ceil_to(R, 8))
    rp = _ceil_to(R, rt)
    xp = _pad2(x, rp, d)
    grid = (rp // rt,)

    def body(x_ref, w_ref, b_ref, o0, o1, o2, o3):
        xv = x_ref[...]
        for k, o in enumerate((o0, o1, o2, o3)):
            o[...] = jnp.dot(xv, w_ref[k], preferred_element_type=F32) + b_ref[k]

    outs = pl.pallas_call(
        body, grid=grid,
        in_specs=[pl.BlockSpec((rt, d), lambda i: (i, 0)),
                  pl.BlockSpec((4, d, d), lambda i: (0, 0, 0)),
                  pl.BlockSpec((4, 1, d), lambda i: (0, 0, 0))],
        out_specs=[pl.BlockSpec((rt, d), lambda i: (i, 0))] * 4,
        out_shape=[jax.ShapeDtypeStruct((rp, d), F32)] * 4,
    )(xp, ws, bs)
    return outs


def _combine(xup, a_st, s_st):
    R, d = xup.shape
    rt = min(ROW_TILE, _ceil_to(R, 8))
    rp = _ceil_to(R, rt)
    grid = (rp // rt,)

    def body(u_ref, a_ref, s_ref, o_ref, cs_ref, cq_ref):
        a = a_ref[0] + a_ref[1]
        sm = s_ref[0] + s_ref[1]
        t = u_ref[...] + a / (sm + 1e-6)
        o_ref[...] = t
        i = pl.program_id(0)
        rows = i * rt + lax.broadcasted_iota(I32, (rt, d), 0)
        tm = jnp.where(rows < R, t, 0.0)

        @pl.when(i == 0)
        def _():
            cs_ref[...] = jnp.zeros_like(cs_ref)
            cq_ref[...] = jnp.zeros_like(cq_ref)

        cs_ref[...] += jnp.sum(tm, axis=0, keepdims=True)
        cq_ref[...] += jnp.sum(tm * tm, axis=0, keepdims=True)

    outs = pl.pallas_call(
        body, grid=grid,
        in_specs=[pl.BlockSpec((rt, d), lambda i: (i, 0)),
                  pl.BlockSpec((2, rt, d), lambda i: (0, i, 0)),
                  pl.BlockSpec((2, rt, d), lambda i: (0, i, 0))],
        out_specs=[pl.BlockSpec((rt, d), lambda i: (i, 0)),
                   pl.BlockSpec((1, d), lambda i: (0, 0)),
                   pl.BlockSpec((1, d), lambda i: (0, 0))],
        out_shape=[jax.ShapeDtypeStruct((rp, d), F32),
                   jax.ShapeDtypeStruct((1, d), F32),
                   jax.ShapeDtypeStruct((1, d), F32)],
    )(_pad2(xup, rp, d), a_st, s_st)
    return outs[0][:R], outs[1], outs[2]


def _norm_apply(t, mu, rstd, g, be, residual=None, split_t=False):
    if split_t:
        R = t.shape[1]
        d = 2 * t.shape[2]
    else:
        R, d = t.shape
    rt = min(ROW_TILE, _ceil_to(R, 8))
    rp = _ceil_to(R, rt)
    grid = (rp // rt,)
    if split_t:
        assert rp == R
        args = [t]
        in_specs = [pl.BlockSpec((2, rt, d // 2), lambda i: (0, i, 0))]
    else:
        args = [_pad2(t, rp, d)]
        in_specs = [pl.BlockSpec((rt, d), lambda i: (i, 0))]
    for v in (mu, rstd, g, be):
        args.append(v.reshape(1, d))
        in_specs.append(pl.BlockSpec((1, d), lambda i: (0, 0)))
    if residual is not None:
        args.append(_pad2(residual, rp, d))
        in_specs.append(pl.BlockSpec((rt, d), lambda i: (i, 0)))

    def body(*refs):
        t_ref, mu_ref, rstd_ref, g_ref, be_ref = refs[:5]
        rest = list(refs[5:])
        if split_t:
            tv = jnp.concatenate([t_ref[0], t_ref[1]], axis=1)
        else:
            tv = t_ref[...]
        v = (tv - mu_ref[...]) * rstd_ref[...] * g_ref[...] + be_ref[...]
        v = _silu(v)
        if residual is not None:
            v = rest.pop(0)[...] + v
        rest.pop(0)[...] = v

    out = pl.pallas_call(
        body, grid=grid, in_specs=in_specs,
        out_specs=pl.BlockSpec((rt, d), lambda i: (i, 0)),
        out_shape=jax.ShapeDtypeStruct((rp, d), F32))(*args)
    return out[:R]


def _sc_messages(es, ed, bh, yg, src, dst, n_out, te_real):
    tep = src.shape[0]
    assert tep % (2 * NSUB * 16) == 0
    ept = tep // (2 * NSUB)
    nchunks = -(-ept // CHUNK)
    spt = nchunks * CHUNK
    tm = 2 * NSUB * spt
    kb = -(-n_out // BROWS)
    rnp = kb * BROWS
    pad_static = spt - ept
    mesh = plsc.VectorSubcoreMesh(core_axis_name="c", subcore_axis_name="s")
    zeros = jnp.zeros((ZR, D), F32)

    @functools.partial(
        pl.kernel, mesh=mesh,
        out_type=[jax.ShapeDtypeStruct((tm, D), F32),
                  jax.ShapeDtypeStruct((tm, D), F32),
                  jax.ShapeDtypeStruct((tm, D), F32),
                  jax.ShapeDtypeStruct((2, rnp, D), F32),
                  jax.ShapeDtypeStruct((2, rnp, D), F32),
                  jax.ShapeDtypeStruct((2, NSUB, 2, D), F32)],
        scratch_types=[pltpu.VMEM((spt,), I32),
                       pltpu.VMEM((ZR, D), F32),
                       pltpu.VMEM((CHUNK, D), F32),
                       pltpu.VMEM((CHUNK, D), F32),
                       pltpu.VMEM((CHUNK, D), F32),
                       pltpu.VMEM((CHUNK, D), F32),
                       pltpu.VMEM((CHUNK,), I32),
                       pltpu.VMEM((CHUNK,), I32),
                       pltpu.VMEM((CHUNK,), I32),
                       pltpu.VMEM((CHUNK,), I32),
                       pltpu.VMEM((CHUNK,), I32),
                       pltpu.VMEM((D,), F32),
                       pltpu.VMEM((D,), F32),
                       pltpu.VMEM_SHARED((TROWS, D), F32),
                       pltpu.VMEM_SHARED((TROWS, D), F32),
                       pltpu.SemaphoreType.DMA])
    def body(es_hbm, ed_hbm, bh_hbm, yg_hbm, src_hbm, dst_hbm, z_hbm,
             m_hbm, p_hbm, g_hbm, a_hbm, s_hbm, st_hbm,
             dstv, zbuf, es_g, ed_g, bh_g, yg_g,
             eid_c, sb_c, src_b, dst_b, dl_b, stat_s, stat_q,
             a_sp, s_sp, sem):
        c = lax.axis_index("c")
        s = lax.axis_index("s")
        tile = c * NSUB + s
        base = tile * ept
        sbase = tile * spt
        pltpu.sync_copy(dst_hbm.at[pl.ds(base, ept)], dstv.at[pl.ds(0, ept)])
        if pad_static:
            pltpu.sync_copy(dst_hbm.at[pl.ds(base, pad_static)],
                            dstv.at[pl.ds(ept, pad_static)])
        pltpu.sync_copy(z_hbm, zbuf)
        for j in range(D // 16):
            stat_s[pl.ds(16 * j, 16)] = jnp.zeros((16,), F32)
            stat_q[pl.ds(16 * j, 16)] = jnp.zeros((16,), F32)

        def edge_masks(off, v, k):
            iota16 = lax.iota(I32, 16)
            rel = off + v * 16
            pos = rel + iota16
            dv = dstv[pl.ds(rel, 16)]
            real = (pos < ept) & ((base + pos) < te_real)
            inb = real & (dv >= k * BROWS) & (dv < (k + 1) * BROWS)
            dl = jnp.where(inb, dv - k * BROWS, BROWS)
            return rel, pos, dv, dl

        for k in range(kb):
            for zi in range(TROWS // 16 // ZR):
                pltpu.sync_copy(zbuf, a_sp.at[pl.ds(s * (TROWS // 16) + zi * ZR, ZR)])
                pltpu.sync_copy(zbuf, s_sp.at[pl.ds(s * (TROWS // 16) + zi * ZR, ZR)])
            plsc.subcore_barrier()

            if k == 0:
                def chunk_body(ci, carry, last=False):
                    off = ci * CHUNK
                    if last:
                        l1 = ept - (nchunks - 1) * CHUNK
                        pltpu.sync_copy(src_hbm.at[pl.ds(base + off, l1)],
                                        sb_c.at[pl.ds(0, l1)])
                        pltpu.sync_copy(src_hbm.at[pl.ds(base, CHUNK - l1)],
                                        sb_c.at[pl.ds(l1, CHUNK - l1)])
                    else:
                        pltpu.sync_copy(src_hbm.at[pl.ds(base + off, CHUNK)],
                                        sb_c)
                    for v in range(CHUNK // 16):
                        rel, pos, dv, dl = edge_masks(off, v, 0)
                        sv = sb_c[pl.ds(v * 16, 16)]
                        eid = base + jnp.where(pos < ept, pos, pos - ept)
                        eid_c[pl.ds(v * 16, 16)] = eid
                        src_b[pl.ds(v * 16, 16)] = sv
                        dst_b[pl.ds(v * 16, 16)] = dv
                        dl_b[pl.ds(v * 16, 16)] = dl
                    cp1 = pltpu.async_copy(es_hbm.at[src_b], es_g, sem)
                    cp2 = pltpu.async_copy(ed_hbm.at[dst_b], ed_g, sem)
                    cp3 = pltpu.async_copy(bh_hbm.at[src_b], bh_g, sem)
                    cp4 = pltpu.async_copy(yg_hbm.at[eid_c], yg_g, sem)
                    cp1.wait()
                    cp2.wait()
                    cp3.wait()
                    cp4.wait()

                    def row_body(i, rc):
                        for j in range(D // 16):
                            sl = pl.ds(16 * j, 16)
                            m_v = es_g[i, sl] + ed_g[i, sl] + yg_g[i, sl]
                            sg = 1.0 / (1.0 + jnp.exp(-m_v))
                            es_g[i, sl] = m_v
                            p_v = sg * bh_g[i, sl]
                            ed_g[i, sl] = sg
                            bh_g[i, sl] = p_v
                            plsc.addupdate(stat_s.at[sl], m_v)
                            plsc.addupdate(stat_q.at[sl], m_v * m_v)
                        return rc

                    lax.fori_loop(0, CHUNK, row_body, 0)
                    pltpu.sync_copy(es_g, m_hbm.at[pl.ds(sbase + off, CHUNK)])
                    pltpu.sync_copy(bh_g, p_hbm.at[pl.ds(sbase + off, CHUNK)])
                    pltpu.sync_copy(ed_g, g_hbm.at[pl.ds(sbase + off, CHUNK)])
                    pltpu.sync_copy(bh_g, a_sp.at[dl_b], add=True)
                    pltpu.sync_copy(ed_g, s_sp.at[dl_b], add=True)
                    return carry
            else:
                def chunk_body(ci, carry, k=k):
                    off = ci * CHUNK
                    for v in range(CHUNK // 16):
                        rel, pos, dv, dl = edge_masks(off, v, k)
                        dl_b[pl.ds(v * 16, 16)] = dl
                    pltpu.sync_copy(p_hbm.at[pl.ds(sbase + off, CHUNK)], bh_g)
                    pltpu.sync_copy(g_hbm.at[pl.ds(sbase + off, CHUNK)], ed_g)
                    pltpu.sync_copy(bh_g, a_sp.at[dl_b], add=True)
                    pltpu.sync_copy(ed_g, s_sp.at[dl_b], add=True)
                    return carry

            if k == 0 and pad_static:
                lax.fori_loop(0, nchunks - 1, chunk_body, 0)
                chunk_body(nchunks - 1, 0, last=True)
            else:
                lax.fori_loop(0, nchunks, chunk_body, 0)
            plsc.subcore_barrier()
            outbase = k * BROWS + s * (BROWS // 16)
            pltpu.sync_copy(a_sp.at[pl.ds(s * (BROWS // 16), BROWS // 16)],
                            a_hbm.at[c].at[pl.ds(outbase, BROWS // 16)])
            pltpu.sync_copy(s_sp.at[pl.ds(s * (BROWS // 16), BROWS // 16)],
                            s_hbm.at[c].at[pl.ds(outbase, BROWS // 16)])
            plsc.subcore_barrier()

        pltpu.sync_copy(stat_s, st_hbm.at[c, s, 0])
        pltpu.sync_copy(stat_q, st_hbm.at[c, s, 1])

    return body(es, ed, bh, yg, src, dst, zeros)


def _egc(p, x, y, src, dst, n_nodes):
    e_real = y.shape[0]
    d = x.shape[1]
    tep = _ceil_to(e_real, 2 * NSUB * 16)
    srcp = jnp.pad(src, (0, tep - e_real))
    dstp = jnp.pad(dst, (0, tep - e_real))
    ws = jnp.stack([p["src_gate"]["W"], p["dst_gate"]["W"],
                    p["src_update"]["W"], p["dst_update"]["W"]])
    bs = jnp.stack([p["src_gate"]["b"].reshape(1, -1),
                    p["dst_gate"]["b"].reshape(1, -1),
                    p["src_update"]["b"].reshape(1, -1),
                    p["dst_update"]["b"].reshape(1, -1)])
    es, ed, bh, xup = _mm4_split(x, ws, bs)
    yg = _mm(y, p["edge_gate"]["W"], p["edge_gate"]["b"])
    ygp = _pad2(yg, tep, d)
    m_str, _, _, a2, s2, mstats = _sc_messages(
        es, ed, bh, ygp, srcp, dstp, n_nodes, e_real)

    ept = tep // (2 * NSUB)
    spt = -(-ept // CHUNK) * CHUNK
    pad = spt - ept
    m4 = m_str.reshape(2 * NSUB, spt, d)
    m = m4[:, :ept, :].reshape(tep, d)[:e_real]
    ms = jnp.sum(mstats[:, :, 0, :], axis=(0, 1))
    mq = jnp.sum(mstats[:, :, 1, :], axis=(0, 1))
    if pad:
        tails = m4[:, ept:, :]
        ms = ms - jnp.sum(tails, axis=(0, 1))
        mq = mq - jnp.sum(tails * tails, axis=(0, 1))
    if tep != e_real:
        lastreal = e_real - (2 * NSUB - 1) * ept
        apad = m4[2 * NSUB - 1, lastreal:ept, :]
        ms = ms - jnp.sum(apad, axis=0)
        mq = mq - jnp.sum(apad * apad, axis=0)
    mmu, mrstd = _finalize_stats(ms.reshape(1, d), mq.reshape(1, d), e_real)

    rt = min(ROW_TILE, _ceil_to(n_nodes, 8))
    rp = _ceil_to(n_nodes, rt)
    xn, cs, cq = _combine(xup[:n_nodes], a2[:, :rp], s2[:, :rp])
    mu, rstd = _finalize_stats(cs, cq, n_nodes)
    x_out = _norm_apply(xn, mu, rstd, p["bn_nodes"]["g"], p["bn_nodes"]["be"],
                        residual=x)
    y_out = _norm_apply(m, mmu, mrstd, p["bn_edges"]["g"],
                        p["bn_edges"]["be"], residual=y)
    return x_out, y_out


def _rbf(x, vmin, vmax, bins):
    centers = jnp.linspace(vmin, vmax, bins)
    gamma = 1.0 / (centers[1] - centers[0])
    return jnp.exp(-gamma * (x[:, None] - centers) ** 2)


def _mlp_pair(x, p1, p2, n):
    t1, s1, q1 = _mm(x, p1["lin"]["W"], p1["lin"]["b"], stats=True)
    mu1, rstd1 = _finalize_stats(s1, q1, n)
    t2, s2, q2 = _mm(t1, p2["lin"]["W"], p2["lin"]["b"], stats=True,
                     norm=(mu1, rstd1, p1["bn"]["g"], p1["bn"]["be"]))
    mu2, rstd2 = _finalize_stats(s2, q2, n)
    return _norm_apply(t2, mu2, rstd2, p2["bn"]["g"], p2["bn"]["be"])


def kernel(atom_features, r, angle_h, edge_index, lg_edge_index, params):
    n = atom_features.shape[0]
    e = r.shape[0]
    t_ang = angle_h.shape[0]

    pa = params["atom_emb"]
    t0, s0, q0 = _mm(atom_features, pa["lin"]["W"], pa["lin"]["b"], stats=True)
    mu0, rstd0 = _finalize_stats(s0, q0, n)
    x = _norm_apply(t0, mu0, rstd0, pa["bn"]["g"], pa["bn"]["be"])

    bondlength = jnp.sqrt(jnp.sum(r * r, axis=1))
    y = _mlp_pair(_rbf(bondlength, 0.0, 8.0, 80),
                  params["edge_mlp1"], params["edge_mlp2"], e)
    z = _mlp_pair(_rbf(angle_h, -1.0, 1.0, 40),
                  params["ang_mlp1"], params["ang_mlp2"], t_ang)

    src = edge_index[0].astype(I32)
    dst = edge_index[1].astype(I32)
    lsrc = lg_edge_index[0].astype(I32)
    ldst = lg_edge_index[1].astype(I32)
    for lp in params["layers"]:
        x, m = _egc(lp["node"], x, y, src, dst, n)
        y, z = _egc(lp["edge"], m, z, lsrc, ldst, e)

    hpool = jnp.mean(x, axis=0)
    return jnp.squeeze(hpool @ params["fc"]["W"] + params["fc"]["b"])
